# baseline (speedup 1.0000x reference)
import jax
import jax.numpy as jnp
from jax.experimental import pallas as pl
from jax.experimental.pallas import tpu as pltpu
from jax.experimental.pallas import tpu_sc as plsc

def _sc_mesh():
    return plsc.VectorSubcoreMesh(core_axis_name="c", subcore_axis_name="s")

_D = 1024
_DFF = 2048
_E = 8
_K = 2
_EPAD = 128
_TG = 256
_T = 256
_NSUB = 32
_WROWS = 32
_NEG = -1e30


def _gate_body(h_ref, gw_ref, gb_ref, aw_ref, wt_ref, cnt_ref):
    x = h_ref[...].astype(jnp.bfloat16)
    s = jnp.dot(x, gw_ref[...], preferred_element_type=jnp.float32)
    s = s + gb_ref[...]
    lane = jax.lax.broadcasted_iota(jnp.int32, s.shape, 1)
    a1 = jnp.argmax(s, axis=-1)
    m1 = jnp.max(s, axis=-1, keepdims=True)
    s2 = jnp.where(lane == a1[:, None], _NEG, s)
    a2 = jnp.argmax(s2, axis=-1)
    m2 = jnp.max(s2, axis=-1, keepdims=True)
    e21 = jnp.exp(m2 - m1)
    w1 = 1.0 / (1.0 + e21)
    w2 = e21 / (1.0 + e21)
    aw_ref[0, :, 0:1] = a1[:, None]
    aw_ref[0, :, 1:2] = a2[:, None]
    wt_ref[0, :, 0:1] = w1
    wt_ref[0, :, 1:2] = w2
    oh = ((lane == a1[:, None]) | (lane == a2[:, None])).astype(jnp.float32)
    z = jnp.sum(oh, axis=0, keepdims=True)

    @pl.when(pl.program_id(0) == 0)
    def _():
        cnt_ref[...] = z

    @pl.when(pl.program_id(0) != 0)
    def _():
        cnt_ref[...] = cnt_ref[...] + z


def _gate(h, gwp, gbp):
    n = h.shape[0]
    ntg = n // _TG
    return pl.pallas_call(
        _gate_body,
        grid=(ntg,),
        in_specs=[
            pl.BlockSpec((_TG, _D), lambda t: (t, 0)),
            pl.BlockSpec((_D, _EPAD), lambda t: (0, 0)),
            pl.BlockSpec((1, _EPAD), lambda t: (0, 0)),
        ],
        out_specs=[
            pl.BlockSpec((1, _TG, _K), lambda t: (t, 0, 0)),
            pl.BlockSpec((1, _TG, _K), lambda t: (t, 0, 0)),
            pl.BlockSpec((1, _EPAD), lambda t: (0, 0)),
        ],
        out_shape=[
            jax.ShapeDtypeStruct((ntg, _TG, _K), jnp.int32),
            jax.ShapeDtypeStruct((ntg, _TG, _K), jnp.float32),
            jax.ShapeDtypeStruct((1, _EPAD), jnp.float32),
        ],
    )(h, gwp, gbp)


def _dst_body(aw_ref, rof_ref, dd_ref, run_ref):
    @pl.when(pl.program_id(0) == 0)
    def _():
        run_ref[...] = jnp.zeros_like(run_ref)

    a1 = aw_ref[0, :, 0]
    a2 = aw_ref[0, :, 1]
    lane = jax.lax.broadcasted_iota(jnp.int32, (_TG, _EPAD), 1)
    oh1 = (lane == a1[:, None]).astype(jnp.float32)
    oh2 = (lane == a2[:, None]).astype(jnp.float32)
    r_io = jax.lax.broadcasted_iota(jnp.int32, (_TG, _TG), 0)
    c_io = jax.lax.broadcasted_iota(jnp.int32, (_TG, _TG), 1)
    tri = (r_io > c_io).astype(jnp.bfloat16)
    rank1 = jnp.dot(tri, oh1.astype(jnp.bfloat16),
                    preferred_element_type=jnp.float32)
    col1 = jnp.sum(oh1, axis=0, keepdims=True)
    rank2 = jnp.dot(tri, oh2.astype(jnp.bfloat16),
                    preferred_element_type=jnp.float32) + col1
    base = run_ref[...] + rof_ref[...]
    d1 = jnp.sum((rank1 + base) * oh1, axis=1)
    d2 = jnp.sum((rank2 + base) * oh2, axis=1)
    dd_ref[0, :, 0:1] = d1.astype(jnp.int32)[:, None]
    dd_ref[0, :, 1:2] = d2.astype(jnp.int32)[:, None]
    run_ref[...] = run_ref[...] + col1 + jnp.sum(oh2, axis=0, keepdims=True)


def _dst(aw, rof):
    ntg = aw.shape[0]
    return pl.pallas_call(
        _dst_body,
        grid=(ntg,),
        in_specs=[
            pl.BlockSpec((1, _TG, _K), lambda t: (t, 0, 0)),
            pl.BlockSpec((1, _EPAD), lambda t: (0, 0)),
        ],
        out_specs=pl.BlockSpec((1, _TG, _K), lambda t: (t, 0, 0)),
        out_shape=jax.ShapeDtypeStruct((ntg, _TG, _K), jnp.int32),
        scratch_shapes=[pltpu.VMEM((1, _EPAD), jnp.float32)],
    )(aw, rof)


def _as32(a):
    n, d = a.shape
    return jax.lax.bitcast_convert_type(a.reshape(n, d // 2, 2), jnp.int32)


def _from32(a, dtype):
    n, d2 = a.shape
    return jax.lax.bitcast_convert_type(a, dtype).reshape(n, d2 * 2)


def _gather_rows(src, idx):
    if src.dtype == jnp.bfloat16:
        return _from32(_gather_rows(_as32(src), idx), jnp.bfloat16)
    n = idx.shape[0]
    d = src.shape[1]
    chunk = n // _NSUB
    nw = chunk // _WROWS
    i2 = idx.reshape(_NSUB, chunk)

    @pl.kernel(out_type=jax.ShapeDtypeStruct((n, d), src.dtype),
               mesh=_sc_mesh(),
               scratch_types=[pltpu.VMEM((1, chunk), jnp.int32),
                              pltpu.VMEM((_WROWS, d), src.dtype),
                              pltpu.VMEM((_WROWS, d), src.dtype),
                              pltpu.SemaphoreType.DMA,
                              pltpu.SemaphoreType.DMA,
                              pltpu.SemaphoreType.DMA])
    def k(x_hbm, i_hbm, o_hbm, ivmem, buf0, buf1, isem, sem0, sem1):
        c = jax.lax.axis_index("c")
        s = jax.lax.axis_index("s")
        u = c * 16 + s
        base = u * chunk
        pltpu.async_copy(i_hbm.at[pl.ds(u, 1)], ivmem, isem).wait()
        bufs = (buf0, buf1)
        sems = (sem0, sem1)
        pending = [None, None]
        for w in range(nw):
            b = w % 2
            if pending[b] is not None:
                pending[b].wait()
            pltpu.sync_copy(x_hbm.at[ivmem.at[0, pl.ds(w * _WROWS, _WROWS)]],
                            bufs[b])
            pending[b] = pltpu.async_copy(
                bufs[b], o_hbm.at[pl.ds(base + w * _WROWS, _WROWS)], sems[b])
        for cp in pending:
            if cp is not None:
                cp.wait()

    return k(src, i2)


def _scatter_rows(src, idx, npad):
    if src.dtype == jnp.bfloat16:
        return _from32(_scatter_rows(_as32(src), idx, npad), jnp.bfloat16)
    n = src.shape[0]
    d = src.shape[1]
    chunk = (_K * n) // _NSUB
    nw = chunk // _WROWS
    i2 = idx.reshape(_NSUB, chunk)

    @pl.kernel(out_type=jax.ShapeDtypeStruct((npad, d), src.dtype),
               mesh=_sc_mesh(),
               scratch_types=[pltpu.VMEM((1, chunk), jnp.int32),
                              pltpu.VMEM((_WROWS, d), src.dtype),
                              pltpu.VMEM((_WROWS, d), src.dtype),
                              pltpu.SemaphoreType.DMA,
                              pltpu.SemaphoreType.DMA,
                              pltpu.SemaphoreType.DMA])
    def k(x_hbm, i_hbm, o_hbm, ivmem, buf0, buf1, isem, sem0, sem1):
        c = jax.lax.axis_index("c")
        s = jax.lax.axis_index("s")
        u = c * 16 + s
        src_base = (u % 16) * chunk
        pltpu.async_copy(i_hbm.at[pl.ds(u, 1)], ivmem, isem).wait()
        bufs = (buf0, buf1)
        sems = (sem0, sem1)
        loads = [None, None]
        loads[0] = pltpu.async_copy(
            x_hbm.at[pl.ds(src_base, _WROWS)], bufs[0], sems[0])
        for w in range(nw):
            b = w % 2
            loads[b].wait()
            if w + 1 < nw:
                nb = (w + 1) % 2
                loads[nb] = pltpu.async_copy(
                    x_hbm.at[pl.ds(src_base + (w + 1) * _WROWS, _WROWS)],
                    bufs[nb], sems[nb])
            pltpu.sync_copy(bufs[b],
                            o_hbm.at[ivmem.at[0, pl.ds(w * _WROWS, _WROWS)]])

    return k(src, i2)


def _ffn_body(eid_ref, x_ref, w1_ref, b1_ref, w2_ref, b2_ref, y_ref):
    x = x_ref[...].astype(jnp.bfloat16)
    hh = jnp.dot(x, w1_ref[0], preferred_element_type=jnp.float32)
    hh = hh + b1_ref[0]
    hh = jax.nn.gelu(hh)
    y = jnp.dot(hh.astype(jnp.bfloat16), w2_ref[0],
                preferred_element_type=jnp.float32)
    y_ref[...] = y + b2_ref[0]


def _ffn(eid, x, w1, b1r, w2, b2r):
    npad = x.shape[0]
    nt = npad // _T
    grid_spec = pltpu.PrefetchScalarGridSpec(
        num_scalar_prefetch=1,
        grid=(nt,),
        in_specs=[
            pl.BlockSpec((_T, _D), lambda t, eid: (t, 0)),
            pl.BlockSpec((1, _D, _DFF), lambda t, eid: (eid[t], 0, 0)),
            pl.BlockSpec((1, 1, _DFF), lambda t, eid: (eid[t], 0, 0)),
            pl.BlockSpec((1, _DFF, _D), lambda t, eid: (eid[t], 0, 0)),
            pl.BlockSpec((1, 1, _D), lambda t, eid: (eid[t], 0, 0)),
        ],
        out_specs=pl.BlockSpec((_T, _D), lambda t, eid: (t, 0)),
    )
    return pl.pallas_call(
        _ffn_body,
        grid_spec=grid_spec,
        out_shape=jax.ShapeDtypeStruct((npad, _D), jnp.float32),
    )(eid, x, w1, b1r, w2, b2r)


def _combine_body(y0_ref, y1_ref, w0_ref, w1_ref, o_ref):
    o_ref[...] = (y0_ref[...] * w0_ref[...]
                  + y1_ref[...] * w1_ref[...]).astype(o_ref.dtype)


def _combine(ysel, w0, w1, n, out_dtype):
    ntg = n // _TG
    return pl.pallas_call(
        _combine_body,
        grid=(ntg,),
        in_specs=[
            pl.BlockSpec((_TG, _D), lambda t: (t, 0)),
            pl.BlockSpec((_TG, _D), lambda t: (t + ntg, 0)),
            pl.BlockSpec((_TG, 1), lambda t: (t, 0)),
            pl.BlockSpec((_TG, 1), lambda t: (t, 0)),
        ],
        out_specs=pl.BlockSpec((_TG, _D), lambda t: (t, 0)),
        out_shape=jax.ShapeDtypeStruct((n, _D), out_dtype),
    )(ysel, ysel, w0, w1)


def _layer(h, gwp, gbp, w1, b1r, w2, b2r, out_dtype):
    n = h.shape[0]
    npad = n * _K + _E * _T
    nt = npad // _T
    aw, wt, cnt = _gate(h, gwp, gbp)
    cnt_i = cnt[0, :_E].astype(jnp.int32)
    tiles_e = (cnt_i + _T - 1) // _T
    tile_off = jnp.concatenate(
        [jnp.zeros((1,), jnp.int32), jnp.cumsum(tiles_e)])
    rof = jnp.pad((tile_off[:_E] * _T).astype(jnp.float32),
                  (0, _EPAD - _E)).reshape(1, _EPAD)
    tile_ids = jnp.arange(nt, dtype=jnp.int32)
    expert_of_tile = jnp.minimum(
        jnp.sum((tile_ids[:, None] >= tile_off[None, 1:]).astype(jnp.int32),
                axis=1),
        _E - 1).astype(jnp.int32)
    dd = _dst(aw, rof)
    ddf = dd.reshape(n, _K)
    pair_idx = jnp.transpose(ddf).reshape(_K * n)
    x = _scatter_rows(h, pair_idx, npad)
    y = _ffn(expert_of_tile, x, w1, b1r, w2, b2r)
    ysel = _gather_rows(y, pair_idx)
    wtf = wt.reshape(n, _K)
    return _combine(ysel, wtf[:, 0:1], wtf[:, 1:2], n, out_dtype)


def kernel(input_ids, emb, gate_W, gate_b, W1, b1, W2, b2):
    B, L = input_ids.shape
    n = B * L
    ids = input_ids.reshape(n).astype(jnp.int32)
    h = _gather_rows(emb, ids)
    nblocks = gate_W.shape[0]
    gwp = jnp.pad(gate_W, ((0, 0), (0, 0),
                           (0, _EPAD - _E))).astype(jnp.bfloat16)
    gbp = jnp.pad(gate_b, ((0, 0), (0, _EPAD - _E)),
                  constant_values=_NEG).reshape(nblocks, 1, _EPAD)
    w1bf = W1.astype(jnp.bfloat16)
    w2bf = W2.astype(jnp.bfloat16)
    b1r = b1.reshape(nblocks, _E, 1, _DFF)
    b2r = b2.reshape(nblocks, _E, 1, _D)
    for li in range(nblocks):
        odt = jnp.float32 if li == nblocks - 1 else jnp.bfloat16
        h = _layer(h, gwp[li], gbp[li], w1bf[li], b1r[li], w2bf[li], b2r[li],
                   odt)
    return h.reshape(B, L, _D)


# speedup vs baseline: 1.3892x; 1.3892x over previous
import jax
import jax.numpy as jnp
from jax.experimental import pallas as pl
from jax.experimental.pallas import tpu as pltpu
from jax.experimental.pallas import tpu_sc as plsc

def _sc_mesh():
    return plsc.VectorSubcoreMesh(core_axis_name="c", subcore_axis_name="s")

_D = 1024
_DFF = 2048
_E = 8
_K = 2
_EPAD = 128
_TG = 256
_T = 256
_NSUB = 32
_WROWS = 32
_NEG = -1e30


def _gate_body(h_ref, gw_ref, gb_ref, aw_ref, wt_ref, cnt_ref):
    x = h_ref[...].astype(jnp.bfloat16)
    s = jnp.dot(x, gw_ref[...], preferred_element_type=jnp.float32)
    s = s + gb_ref[...]
    lane = jax.lax.broadcasted_iota(jnp.int32, s.shape, 1)
    a1 = jnp.argmax(s, axis=-1)
    m1 = jnp.max(s, axis=-1, keepdims=True)
    s2 = jnp.where(lane == a1[:, None], _NEG, s)
    a2 = jnp.argmax(s2, axis=-1)
    m2 = jnp.max(s2, axis=-1, keepdims=True)
    e21 = jnp.exp(m2 - m1)
    w1 = 1.0 / (1.0 + e21)
    w2 = e21 / (1.0 + e21)
    aw_ref[0, :, 0:1] = a1[:, None]
    aw_ref[0, :, 1:2] = a2[:, None]
    wt_ref[0, :, 0:1] = w1
    wt_ref[0, :, 1:2] = w2
    oh = ((lane == a1[:, None]) | (lane == a2[:, None])).astype(jnp.float32)
    z = jnp.sum(oh, axis=0, keepdims=True)

    @pl.when(pl.program_id(0) == 0)
    def _():
        cnt_ref[...] = z

    @pl.when(pl.program_id(0) != 0)
    def _():
        cnt_ref[...] = cnt_ref[...] + z


def _gate(h, gwp, gbp):
    n = h.shape[0]
    ntg = n // _TG
    return pl.pallas_call(
        _gate_body,
        grid=(ntg,),
        in_specs=[
            pl.BlockSpec((_TG, _D), lambda t: (t, 0)),
            pl.BlockSpec((_D, _EPAD), lambda t: (0, 0)),
            pl.BlockSpec((1, _EPAD), lambda t: (0, 0)),
        ],
        out_specs=[
            pl.BlockSpec((1, _TG, _K), lambda t: (t, 0, 0)),
            pl.BlockSpec((1, _TG, _K), lambda t: (t, 0, 0)),
            pl.BlockSpec((1, _EPAD), lambda t: (0, 0)),
        ],
        out_shape=[
            jax.ShapeDtypeStruct((ntg, _TG, _K), jnp.int32),
            jax.ShapeDtypeStruct((ntg, _TG, _K), jnp.float32),
            jax.ShapeDtypeStruct((1, _EPAD), jnp.float32),
        ],
    )(h, gwp, gbp)


def _dst_body(aw_ref, rof_ref, dd_ref, run_ref):
    @pl.when(pl.program_id(0) == 0)
    def _():
        run_ref[...] = jnp.zeros_like(run_ref)

    a1 = aw_ref[0, :, 0]
    a2 = aw_ref[0, :, 1]
    lane = jax.lax.broadcasted_iota(jnp.int32, (_TG, _EPAD), 1)
    oh1 = (lane == a1[:, None]).astype(jnp.float32)
    oh2 = (lane == a2[:, None]).astype(jnp.float32)
    r_io = jax.lax.broadcasted_iota(jnp.int32, (_TG, _TG), 0)
    c_io = jax.lax.broadcasted_iota(jnp.int32, (_TG, _TG), 1)
    tri = (r_io > c_io).astype(jnp.bfloat16)
    rank1 = jnp.dot(tri, oh1.astype(jnp.bfloat16),
                    preferred_element_type=jnp.float32)
    col1 = jnp.sum(oh1, axis=0, keepdims=True)
    rank2 = jnp.dot(tri, oh2.astype(jnp.bfloat16),
                    preferred_element_type=jnp.float32) + col1
    base = run_ref[...] + rof_ref[...]
    d1 = jnp.sum((rank1 + base) * oh1, axis=1)
    d2 = jnp.sum((rank2 + base) * oh2, axis=1)
    dd_ref[0, :, 0:1] = d1.astype(jnp.int32)[:, None]
    dd_ref[0, :, 1:2] = d2.astype(jnp.int32)[:, None]
    run_ref[...] = run_ref[...] + col1 + jnp.sum(oh2, axis=0, keepdims=True)


def _dst(aw, rof):
    ntg = aw.shape[0]
    return pl.pallas_call(
        _dst_body,
        grid=(ntg,),
        in_specs=[
            pl.BlockSpec((1, _TG, _K), lambda t: (t, 0, 0)),
            pl.BlockSpec((1, _EPAD), lambda t: (0, 0)),
        ],
        out_specs=pl.BlockSpec((1, _TG, _K), lambda t: (t, 0, 0)),
        out_shape=jax.ShapeDtypeStruct((ntg, _TG, _K), jnp.int32),
        scratch_shapes=[pltpu.VMEM((1, _EPAD), jnp.float32)],
    )(aw, rof)


def _gather_rows(src, idx):
    n = idx.shape[0]
    d = src.shape[1]
    chunk = n // _NSUB
    nw = chunk // _WROWS
    i2 = idx.reshape(_NSUB, chunk)

    @pl.kernel(out_type=jax.ShapeDtypeStruct((n, d), src.dtype),
               mesh=_sc_mesh(),
               scratch_types=[pltpu.VMEM((1, chunk), jnp.int32),
                              pltpu.VMEM((_WROWS, d), src.dtype),
                              pltpu.VMEM((_WROWS, d), src.dtype),
                              pltpu.SemaphoreType.DMA,
                              pltpu.SemaphoreType.DMA,
                              pltpu.SemaphoreType.DMA])
    def k(x_hbm, i_hbm, o_hbm, ivmem, buf0, buf1, isem, sem0, sem1):
        c = jax.lax.axis_index("c")
        s = jax.lax.axis_index("s")
        u = c * 16 + s
        base = u * chunk
        pltpu.async_copy(i_hbm.at[pl.ds(u, 1)], ivmem, isem).wait()
        bufs = (buf0, buf1)
        sems = (sem0, sem1)
        pending = [None, None]
        for w in range(nw):
            b = w % 2
            if pending[b] is not None:
                pending[b].wait()
            pltpu.sync_copy(x_hbm.at[ivmem.at[0, pl.ds(w * _WROWS, _WROWS)]],
                            bufs[b])
            pending[b] = pltpu.async_copy(
                bufs[b], o_hbm.at[pl.ds(base + w * _WROWS, _WROWS)], sems[b])
        for cp in pending:
            if cp is not None:
                cp.wait()

    return k(src, i2)


def _scatter_rows(src, idx, npad):
    n = src.shape[0]
    d = src.shape[1]
    chunk = (_K * n) // _NSUB
    nw = chunk // _WROWS
    i2 = idx.reshape(_NSUB, chunk)

    @pl.kernel(out_type=jax.ShapeDtypeStruct((npad, d), src.dtype),
               mesh=_sc_mesh(),
               scratch_types=[pltpu.VMEM((1, chunk), jnp.int32),
                              pltpu.VMEM((_WROWS, d), src.dtype),
                              pltpu.VMEM((_WROWS, d), src.dtype),
                              pltpu.SemaphoreType.DMA,
                              pltpu.SemaphoreType.DMA,
                              pltpu.SemaphoreType.DMA])
    def k(x_hbm, i_hbm, o_hbm, ivmem, buf0, buf1, isem, sem0, sem1):
        c = jax.lax.axis_index("c")
        s = jax.lax.axis_index("s")
        u = c * 16 + s
        src_base = (u % 16) * chunk
        pltpu.async_copy(i_hbm.at[pl.ds(u, 1)], ivmem, isem).wait()
        bufs = (buf0, buf1)
        sems = (sem0, sem1)
        loads = [None, None]
        loads[0] = pltpu.async_copy(
            x_hbm.at[pl.ds(src_base, _WROWS)], bufs[0], sems[0])
        for w in range(nw):
            b = w % 2
            loads[b].wait()
            if w + 1 < nw:
                nb = (w + 1) % 2
                loads[nb] = pltpu.async_copy(
                    x_hbm.at[pl.ds(src_base + (w + 1) * _WROWS, _WROWS)],
                    bufs[nb], sems[nb])
            pltpu.sync_copy(bufs[b],
                            o_hbm.at[ivmem.at[0, pl.ds(w * _WROWS, _WROWS)]])

    return k(src, i2)


def _ffn_body(eid_ref, x_ref, w1_ref, b1_ref, w2_ref, b2_ref, y_ref):
    x = x_ref[...].astype(jnp.bfloat16)
    hh = jnp.dot(x, w1_ref[0], preferred_element_type=jnp.float32)
    hh = hh + b1_ref[0]
    hh = jax.nn.gelu(hh)
    y = jnp.dot(hh.astype(jnp.bfloat16), w2_ref[0],
                preferred_element_type=jnp.float32)
    y_ref[...] = y + b2_ref[0]


def _ffn(eid, x, w1, b1r, w2, b2r):
    npad = x.shape[0]
    nt = npad // _T
    grid_spec = pltpu.PrefetchScalarGridSpec(
        num_scalar_prefetch=1,
        grid=(nt,),
        in_specs=[
            pl.BlockSpec((_T, _D), lambda t, eid: (t, 0)),
            pl.BlockSpec((1, _D, _DFF), lambda t, eid: (eid[t], 0, 0)),
            pl.BlockSpec((1, 1, _DFF), lambda t, eid: (eid[t], 0, 0)),
            pl.BlockSpec((1, _DFF, _D), lambda t, eid: (eid[t], 0, 0)),
            pl.BlockSpec((1, 1, _D), lambda t, eid: (eid[t], 0, 0)),
        ],
        out_specs=pl.BlockSpec((_T, _D), lambda t, eid: (t, 0)),
    )
    return pl.pallas_call(
        _ffn_body,
        grid_spec=grid_spec,
        out_shape=jax.ShapeDtypeStruct((npad, _D), jnp.float32),
    )(eid, x, w1, b1r, w2, b2r)


def _combine_body(y0_ref, y1_ref, w0_ref, w1_ref, o_ref):
    o_ref[...] = y0_ref[...] * w0_ref[...] + y1_ref[...] * w1_ref[...]


def _combine(ysel, w0, w1, n):
    ntg = n // _TG
    return pl.pallas_call(
        _combine_body,
        grid=(ntg,),
        in_specs=[
            pl.BlockSpec((_TG, _D), lambda t: (t, 0)),
            pl.BlockSpec((_TG, _D), lambda t: (t + ntg, 0)),
            pl.BlockSpec((_TG, 1), lambda t: (t, 0)),
            pl.BlockSpec((_TG, 1), lambda t: (t, 0)),
        ],
        out_specs=pl.BlockSpec((_TG, _D), lambda t: (t, 0)),
        out_shape=jax.ShapeDtypeStruct((n, _D), jnp.float32),
    )(ysel, ysel, w0, w1)


def _layer(h, gwp, gbp, w1, b1r, w2, b2r):
    n = h.shape[0]
    npad = n * _K + _E * _T
    nt = npad // _T
    aw, wt, cnt = _gate(h, gwp, gbp)
    cnt_i = cnt[0, :_E].astype(jnp.int32)
    tiles_e = (cnt_i + _T - 1) // _T
    tile_off = jnp.concatenate(
        [jnp.zeros((1,), jnp.int32), jnp.cumsum(tiles_e)])
    rof = jnp.pad((tile_off[:_E] * _T).astype(jnp.float32),
                  (0, _EPAD - _E)).reshape(1, _EPAD)
    tile_ids = jnp.arange(nt, dtype=jnp.int32)
    expert_of_tile = jnp.minimum(
        jnp.sum((tile_ids[:, None] >= tile_off[None, 1:]).astype(jnp.int32),
                axis=1),
        _E - 1).astype(jnp.int32)
    dd = _dst(aw, rof)
    ddf = dd.reshape(n, _K)
    pair_idx = jnp.transpose(ddf).reshape(_K * n)
    x = _scatter_rows(h, pair_idx, npad)
    y = _ffn(expert_of_tile, x, w1, b1r, w2, b2r)
    ysel = _gather_rows(y, pair_idx)
    wtf = wt.reshape(n, _K)
    return _combine(ysel, wtf[:, 0:1], wtf[:, 1:2], n)


def kernel(input_ids, emb, gate_W, gate_b, W1, b1, W2, b2):
    B, L = input_ids.shape
    n = B * L
    ids = input_ids.reshape(n).astype(jnp.int32)
    h = _gather_rows(emb, ids)
    nblocks = gate_W.shape[0]
    gwp = jnp.pad(gate_W, ((0, 0), (0, 0),
                           (0, _EPAD - _E))).astype(jnp.bfloat16)
    gbp = jnp.pad(gate_b, ((0, 0), (0, _EPAD - _E)),
                  constant_values=_NEG).reshape(nblocks, 1, _EPAD)
    w1bf = W1.astype(jnp.bfloat16)
    w2bf = W2.astype(jnp.bfloat16)
    b1r = b1.reshape(nblocks, _E, 1, _DFF)
    b2r = b2.reshape(nblocks, _E, 1, _D)
    for li in range(nblocks):
        h = _layer(h, gwp[li], gbp[li], w1bf[li], b1r[li], w2bf[li], b2r[li])
    return h.reshape(B, L, _D)
